# baseline (speedup 1.0000x reference)
import jax
import jax.numpy as jnp
import numpy as np
from jax import lax
from jax.experimental import pallas as pl
from jax.experimental.pallas import tpu as pltpu
from jax.experimental.pallas import tpu_sc as plsc

_N = 10000
_E = 160000
_H = 128
_R = 32
_CUT = 5.0

_BN = 1000
_BE = 2000

_NW = 32
_EPW = _E // _NW
_BSC = 40
_NPT = 624
_NTAIL = _N - 16 * _NPT
_ZR = 208
_G_OF_K = (0, 1, 1, 1, 2, 2, 2, 2, 2)


def _silu(x):
    return x / (1.0 + jnp.exp(-x))


def _edge_mlp_body(attr_ref, ew_ref, ws1, b1, ws2, b2,
                   w3a, b3a, w3b, b3b, w3c, b3c, out_ref):
    x = attr_ref[...]
    h1 = _silu(jnp.dot(x, ws1[...], preferred_element_type=jnp.float32) + b1[...])
    h2 = _silu(jnp.dot(h1, ws2[...], preferred_element_type=jnp.float32) + b2[...])
    r = ew_ref[...]
    c = 0.5 * (jnp.cos(r * (np.pi / _CUT)) + 1.0) * (r < _CUT).astype(jnp.float32)
    for ci, (w, b) in enumerate(((w3a, b3a), (w3b, b3b), (w3c, b3c))):
        out_ref[ci] = _silu(
            jnp.dot(h2, w[...], preferred_element_type=jnp.float32) + b[...]) * c


def _prep_body(x_ref, wt0, wt1, wt2, xn_ref, v_ref):
    x = x_ref[...]
    nrm = (x * x).sum(axis=0)
    xn = x / (nrm + 1.0)
    xn_ref[...] = xn
    iv = (xn[0] + xn[4] + xn[8]) * (1.0 / 3.0)
    a01 = 0.5 * (xn[1] - xn[3])
    a02 = 0.5 * (xn[2] - xn[6])
    a12 = 0.5 * (xn[5] - xn[7])
    s00 = xn[0] - iv
    s11 = xn[4] - iv
    s01 = 0.5 * (xn[1] + xn[3])
    s02 = 0.5 * (xn[2] + xn[6])
    s12 = 0.5 * (xn[5] + xn[7])
    w0 = wt0[...]
    w1 = wt1[...]
    w2 = wt2[...]
    dot = lambda a, w: jnp.dot(a, w, preferred_element_type=jnp.float32)
    v_ref[0] = dot(iv, w0)
    v_ref[1] = dot(a01, w1)
    v_ref[2] = dot(a02, w1)
    v_ref[3] = dot(a12, w1)
    v_ref[4] = dot(s00, w2)
    v_ref[5] = dot(s01, w2)
    v_ref[6] = dot(s02, w2)
    v_ref[7] = dot(s11, w2)
    v_ref[8] = dot(s12, w2)


def _sc_msg_kernel(v_hbm, ea_hbm, src_hbm, dst_hbm, out_hbm,
                   srcv, dstv, rows, eav, zrow, shared, sem):
    cid = lax.axis_index("c")
    sid = lax.axis_index("s")
    wid = cid * 16 + sid
    edge_base = wid * _EPW
    row_lo = sid * _NPT

    def zinit(i, c):
        for j in range(_H // 16):
            zrow[i, pl.ds(j * 16, 16)] = jnp.zeros((16,), jnp.float32)
        return c

    lax.fori_loop(0, _ZR, zinit, 0)

    for k in range(9):
        for z in range(_NPT // _ZR):
            pltpu.sync_copy(zrow, shared.at[pl.ds(row_lo + z * _ZR, _ZR)])

        @pl.when(sid == 15)
        def _():
            pltpu.sync_copy(zrow.at[pl.ds(0, _NTAIL)],
                            shared.at[pl.ds(16 * _NPT, _NTAIL)])

        plsc.subcore_barrier()

        def batch(b, carry):
            base = pl.multiple_of(edge_base + b * _BSC, 8)
            pltpu.sync_copy(src_hbm.at[pl.ds(base, _BSC)], srcv)
            pltpu.sync_copy(dst_hbm.at[pl.ds(base, _BSC)], dstv)
            pltpu.async_copy(v_hbm.at[k].at[srcv], rows, sem).wait()
            pltpu.sync_copy(ea_hbm.at[_G_OF_K[k], pl.ds(base, _BSC)], eav)

            def edge(i, c2):
                for j in range(_H // 16):
                    rows[i, pl.ds(j * 16, 16)] = (
                        rows[i, pl.ds(j * 16, 16)]
                        * eav[i, pl.ds(j * 16, 16)])
                return c2

            lax.fori_loop(0, _BSC, edge, 0)
            pltpu.sync_copy(rows, shared.at[dstv], add=True)
            return carry

        lax.fori_loop(0, _EPW // _BSC, batch, 0)
        plsc.subcore_barrier()
        pltpu.sync_copy(shared.at[pl.ds(row_lo, _NPT)],
                        out_hbm.at[cid, k, pl.ds(row_lo, _NPT)])

        @pl.when(sid == 15)
        def _():
            pltpu.sync_copy(shared.at[pl.ds(16 * _NPT, _NTAIL)],
                            out_hbm.at[cid, k, pl.ds(16 * _NPT, _NTAIL)])

        plsc.subcore_barrier()


def _full9(t):
    iv, a01, a02, a12, s00, s01, s02, s11, s12 = t
    return (iv + s00, s01 + a01, s02 + a02,
            s01 - a01, iv + s11, s12 + a12,
            s02 - a02, s12 - a12, iv - s00 - s11)


def _post_body(xn_ref, v_ref, m_ref, wt3, wt4, wt5, out_ref):
    vv = v_ref[...]
    vm = m_ref[0] + m_ref[1]
    Y = _full9(tuple(vv[k] for k in range(9)))
    M = _full9(tuple(vm[k] for k in range(9)))
    y = [[Y[0], Y[1], Y[2]], [Y[3], Y[4], Y[5]], [Y[6], Y[7], Y[8]]]
    m = [[M[0], M[1], M[2]], [M[3], M[4], M[5]], [M[6], M[7], M[8]]]
    c = [[None] * 3 for _ in range(3)]
    for i in range(3):
        for j in range(3):
            acc = m[i][0] * y[0][j] + y[i][0] * m[0][j]
            for kk in (1, 2):
                acc = acc + m[i][kk] * y[kk][j] + y[i][kk] * m[kk][j]
            c[i][j] = acc
    nrm = None
    for i in range(3):
        for j in range(3):
            t = c[i][j] * c[i][j]
            nrm = t if nrm is None else nrm + t
    inv = 1.0 / (nrm + 1.0)
    ivc = (c[0][0] + c[1][1] + c[2][2]) * (1.0 / 3.0)
    a01c = 0.5 * (c[0][1] - c[1][0])
    a02c = 0.5 * (c[0][2] - c[2][0])
    a12c = 0.5 * (c[1][2] - c[2][1])
    s00c = c[0][0] - ivc
    s11c = c[1][1] - ivc
    s01c = 0.5 * (c[0][1] + c[1][0])
    s02c = 0.5 * (c[0][2] + c[2][0])
    s12c = 0.5 * (c[1][2] + c[2][1])
    w3 = wt3[...]
    w4 = wt4[...]
    w5 = wt5[...]
    dot = lambda a, w: jnp.dot(a * inv, w, preferred_element_type=jnp.float32)
    D = _full9((dot(ivc, w3),
                dot(a01c, w4), dot(a02c, w4), dot(a12c, w4),
                dot(s00c, w5), dot(s01c, w5), dot(s02c, w5),
                dot(s11c, w5), dot(s12c, w5)))
    d = [[D[0], D[1], D[2]], [D[3], D[4], D[5]], [D[6], D[7], D[8]]]
    xn = xn_ref[...]
    for i in range(3):
        for j in range(3):
            acc = d[i][j]
            for kk in range(3):
                acc = acc + d[i][kk] * d[kk][j]
            out_ref[i * 3 + j] = xn[i * 3 + j] + acc


def kernel(X, edge_index, edge_weight, edge_attr,
           Ws1, bs1, Ws2, bs2, Ws3, bs3, Wt0, Wt1, Wt2, Wt3, Wt4, Wt5):
    f32 = jnp.float32
    X9 = X.reshape(_N, _H, 9).transpose(2, 0, 1)
    ei = edge_index.astype(jnp.int32)
    ew2 = edge_weight.reshape(_E, 1)
    b1 = bs1.reshape(1, _H)
    b2 = bs2.reshape(1, 2 * _H)
    w3s = [Ws3[:, ci::3] for ci in range(3)]
    b3s = [bs3[ci::3].reshape(1, _H) for ci in range(3)]

    full = lambda *shape: pl.BlockSpec(shape, lambda g: (0,) * len(shape))

    EA = pl.pallas_call(
        _edge_mlp_body,
        grid=(_E // _BE,),
        in_specs=[
            pl.BlockSpec((_BE, _R), lambda g: (g, 0)),
            pl.BlockSpec((_BE, 1), lambda g: (g, 0)),
            full(_R, _H), full(1, _H),
            full(_H, 2 * _H), full(1, 2 * _H),
            full(2 * _H, _H), full(1, _H),
            full(2 * _H, _H), full(1, _H),
            full(2 * _H, _H), full(1, _H),
        ],
        out_specs=pl.BlockSpec((3, _BE, _H), lambda g: (0, g, 0)),
        out_shape=jax.ShapeDtypeStruct((3, _E, _H), f32),
    )(edge_attr, ew2, Ws1, b1, Ws2, b2,
      w3s[0], b3s[0], w3s[1], b3s[1], w3s[2], b3s[2])

    Xn9, V9 = pl.pallas_call(
        _prep_body,
        grid=(_N // _BN,),
        in_specs=[
            pl.BlockSpec((9, _BN, _H), lambda g: (0, g, 0)),
            full(_H, _H), full(_H, _H), full(_H, _H),
        ],
        out_specs=[
            pl.BlockSpec((9, _BN, _H), lambda g: (0, g, 0)),
            pl.BlockSpec((9, _BN, _H), lambda g: (0, g, 0)),
        ],
        out_shape=[
            jax.ShapeDtypeStruct((9, _N, _H), f32),
            jax.ShapeDtypeStruct((9, _N, _H), f32),
        ],
    )(X9, Wt0, Wt1, Wt2)

    MSG2 = pl.kernel(
        _sc_msg_kernel,
        mesh=plsc.VectorSubcoreMesh(core_axis_name="c", subcore_axis_name="s"),
        out_type=jax.ShapeDtypeStruct((2, 9, _N, _H), f32),
        scratch_types=[
            pltpu.VMEM((_BSC,), jnp.int32),
            pltpu.VMEM((_BSC,), jnp.int32),
            pltpu.VMEM((_BSC, _H), f32),
            pltpu.VMEM((_BSC, _H), f32),
            pltpu.VMEM((_ZR, _H), f32),
            pltpu.VMEM_SHARED((_N, _H), f32),
            pltpu.SemaphoreType.DMA,
        ],
    )(V9, EA, ei[0], ei[1])

    OUT9 = pl.pallas_call(
        _post_body,
        grid=(_N // _BN,),
        in_specs=[
            pl.BlockSpec((9, _BN, _H), lambda g: (0, g, 0)),
            pl.BlockSpec((9, _BN, _H), lambda g: (0, g, 0)),
            pl.BlockSpec((2, 9, _BN, _H), lambda g: (0, 0, g, 0)),
            full(_H, _H), full(_H, _H), full(_H, _H),
        ],
        out_specs=pl.BlockSpec((9, _BN, _H), lambda g: (0, g, 0)),
        out_shape=jax.ShapeDtypeStruct((9, _N, _H), f32),
    )(Xn9, V9, MSG2, Wt3, Wt4, Wt5)

    return OUT9.transpose(1, 2, 0).reshape(_N, _H, 3, 3)
